# Initial kernel scaffold; baseline (speedup 1.0000x reference)
import math

import jax
import jax.numpy as jnp
from jax import lax
from jax.experimental import pallas as pl
from jax.experimental.pallas import tpu as pltpu
from jax.experimental.pallas import tpu_sc as plsc

_VOCAB = 1000
_VPAD = 1024
_D = 128
_B, _N = 64, 32
_K = 100
_BN = _B * _N
_KPAD = 128
_LOGK = math.log(_K)

_NC, _NS = 2, 16
_NW = _NC * _NS
_TPW = _BN // _NW
_NPW = 8


def _sc_gather_body(tgt_hbm, noise_hbm, emb_hbm, bias_hbm,
                    te_out, ne_out, bt_out, bn_out,
                    idx_v, rows_v, btbl_v, bval_v,
                    nidx_v, nrows_v, nbidx_v, nbval_v, sem):
    wid = lax.axis_index("s") * _NC + lax.axis_index("c")
    base = wid * _TPW
    pltpu.sync_copy(tgt_hbm.at[pl.ds(base, _TPW)], idx_v)
    pltpu.async_copy(emb_hbm.at[idx_v], rows_v, sem).wait()
    pltpu.sync_copy(rows_v, te_out.at[pl.ds(base, _TPW)])
    pltpu.sync_copy(bias_hbm, btbl_v)
    for j in range(_TPW // 16):
        idx16 = idx_v[pl.ds(j * 16, 16)]
        bval_v[pl.ds(j * 16, 16)] = plsc.load_gather(btbl_v, [idx16])
    pltpu.sync_copy(bval_v, bt_out.at[pl.ds(base, _TPW)])

    @pl.when(wid < _KPAD // _NPW)
    def _():
        nb = wid * _NPW
        pltpu.sync_copy(noise_hbm.at[pl.ds(nb, _NPW)], nidx_v)
        pltpu.async_copy(emb_hbm.at[nidx_v], nrows_v, sem).wait()
        pltpu.sync_copy(nrows_v, ne_out.at[pl.ds(nb, _NPW)])

    @pl.when(wid < _KPAD // 16)
    def _():
        mb = wid * 16
        pltpu.sync_copy(noise_hbm.at[pl.ds(mb, 16)], nbidx_v)
        nbval_v[...] = plsc.load_gather(btbl_v, [nbidx_v[...]])
        pltpu.sync_copy(nbval_v, bn_out.at[pl.ds(mb, 16)])


_sc_gather = pl.kernel(
    _sc_gather_body,
    out_type=(
        jax.ShapeDtypeStruct((_BN, _D), jnp.float32),
        jax.ShapeDtypeStruct((_KPAD, _D), jnp.float32),
        jax.ShapeDtypeStruct((_BN,), jnp.float32),
        jax.ShapeDtypeStruct((_KPAD,), jnp.float32),
    ),
    mesh=plsc.VectorSubcoreMesh(core_axis_name="c", subcore_axis_name="s"),
    scratch_types=[
        pltpu.VMEM((_TPW,), jnp.int32),
        pltpu.VMEM((_TPW, _D), jnp.float32),
        pltpu.VMEM((_VPAD,), jnp.float32),
        pltpu.VMEM((_TPW,), jnp.float32),
        pltpu.VMEM((_NPW,), jnp.int32),
        pltpu.VMEM((_NPW, _D), jnp.float32),
        pltpu.VMEM((16,), jnp.int32),
        pltpu.VMEM((16,), jnp.float32),
        pltpu.SemaphoreType.DMA,
    ],
)


def _tc_body(x_ref, te_ref, ne_ref, bt_ref, bn_ref, out_ref):
    x = x_ref[...]
    t = jnp.sum(x * te_ref[...], axis=1, keepdims=True) + bt_ref[...] - _LOGK
    s = lax.dot_general(x, ne_ref[...], (((1,), (1,)), ((), ())),
                        preferred_element_type=jnp.float32)
    xn = s + bn_ref[...] - _LOGK
    sp_n = jnp.maximum(xn, 0.0) + jnp.log1p(jnp.exp(-jnp.abs(xn)))
    col = lax.broadcasted_iota(jnp.int32, sp_n.shape, 1)
    sp_n = jnp.where(col < _K, sp_n, 0.0)
    sp_t = jnp.maximum(-t, 0.0) + jnp.log1p(jnp.exp(-jnp.abs(t)))
    out_ref[0, 0] = (jnp.sum(sp_n) + jnp.sum(sp_t)) * (1.0 / _BN)


def kernel(target, input, emb, bias, noise_samples):
    tgt = target.reshape(_BN).astype(jnp.int32)
    noise = jnp.pad(noise_samples.reshape(_K).astype(jnp.int32),
                    (0, _KPAD - _K))
    bias_pad = jnp.pad(bias.astype(jnp.float32), (0, _VPAD - _VOCAB))
    x = input.reshape(_BN, _D)
    te, ne, bt, bn = _sc_gather(tgt, noise, emb, bias_pad)
    out = pl.pallas_call(
        _tc_body,
        out_shape=jax.ShapeDtypeStruct((1, 1), jnp.float32),
    )(x, te, ne, bt.reshape(_BN, 1), bn.reshape(1, _KPAD))
    return out[0, 0]


# speedup vs baseline: 112.5937x; 112.5937x over previous
import functools
import math

import jax
import jax.numpy as jnp
from jax import lax
from jax.experimental import pallas as pl
from jax.experimental.pallas import tpu as pltpu
from jax.experimental.pallas import tpu_sc as plsc

_VOCAB = 1000
_VPAD = 1024
_D = 128
_B, _N = 64, 32
_K = 100
_BN = _B * _N
_KPAD = 128
_LOGK = math.log(_K)

_NC, _NS = 2, 16
_NW = _NC * _NS
_TPW = _BN // _NW
_NPW = 8


def _sc_gather_body(tgt_hbm, noise_hbm, emb_hbm,
                    te_out, ne_out,
                    idx_v, rows_v, nidx_v, nrows_v, sem):
    wid = lax.axis_index("s") * _NC + lax.axis_index("c")
    base = wid * _TPW
    pltpu.sync_copy(tgt_hbm.at[pl.ds(base, _TPW)], idx_v)
    pltpu.async_copy(emb_hbm.at[idx_v], rows_v, sem).wait()
    pltpu.sync_copy(rows_v, te_out.at[pl.ds(base, _TPW)])

    @pl.when(wid < _KPAD // _NPW)
    def _():
        nb = wid * _NPW
        pltpu.sync_copy(noise_hbm.at[pl.ds(nb, _NPW)], nidx_v)
        pltpu.async_copy(emb_hbm.at[nidx_v], nrows_v, sem).wait()
        pltpu.sync_copy(nrows_v, ne_out.at[pl.ds(nb, _NPW)])


@functools.cache
def _make_sc_gather():
  return pl.kernel(
    _sc_gather_body,
    out_type=(
        jax.ShapeDtypeStruct((_BN, _D), jnp.float32),
        jax.ShapeDtypeStruct((_KPAD, _D), jnp.float32),
    ),
    mesh=plsc.VectorSubcoreMesh(core_axis_name="c", subcore_axis_name="s",
                                num_cores=_NC, num_subcores=_NS),
    scratch_types=[
        pltpu.VMEM((_TPW,), jnp.int32),
        pltpu.VMEM((_TPW, _D), jnp.float32),
        pltpu.VMEM((_NPW,), jnp.int32),
        pltpu.VMEM((_NPW, _D), jnp.float32),
        pltpu.SemaphoreType.DMA,
    ],
  )


def _tc_body(x_ref, te_ref, ne_ref, tgt_ref, nz_ref, brow_ref, bcol_ref,
             out_ref):
    x = x_ref[...]
    tgt = tgt_ref[...]
    vid_t = lax.broadcasted_iota(jnp.int32, (_BN, _VPAD), 1)
    brow = brow_ref[...]
    bt = jnp.sum(jnp.where(vid_t == tgt, brow, 0.0), axis=1, keepdims=True)
    nz = nz_ref[...]
    vid_n = lax.broadcasted_iota(jnp.int32, (_VPAD, _KPAD), 0)
    bcol = bcol_ref[...]
    bn = jnp.sum(jnp.where(vid_n == nz, bcol, 0.0), axis=0, keepdims=True)

    t = jnp.sum(x * te_ref[...], axis=1, keepdims=True) + bt - _LOGK
    s = lax.dot_general(x, ne_ref[...], (((1,), (1,)), ((), ())),
                        preferred_element_type=jnp.float32)
    xn = s + bn - _LOGK
    sp_n = jnp.maximum(xn, 0.0) + jnp.log1p(jnp.exp(-jnp.abs(xn)))
    col = lax.broadcasted_iota(jnp.int32, sp_n.shape, 1)
    sp_n = jnp.where(col < _K, sp_n, 0.0)
    sp_t = jnp.maximum(-t, 0.0) + jnp.log1p(jnp.exp(-jnp.abs(t)))
    total = (jnp.sum(sp_n) + jnp.sum(sp_t)) * (1.0 / _BN)
    out_ref[...] = jnp.broadcast_to(total, (1, 1))


def kernel(target, input, emb, bias, noise_samples):
    tgt = target.reshape(_BN).astype(jnp.int32)
    noise = jnp.pad(noise_samples.reshape(_K).astype(jnp.int32),
                    (0, _KPAD - _K))
    bias_pad = jnp.pad(bias.astype(jnp.float32), (0, _VPAD - _VOCAB))
    x = input.reshape(_BN, _D)
    te, ne = _make_sc_gather()(tgt, noise, emb)
    out = pl.pallas_call(
        _tc_body,
        out_shape=jax.ShapeDtypeStruct((1, 1), jnp.float32),
    )(x, te, ne, tgt.reshape(_BN, 1), noise.reshape(1, _KPAD),
      bias_pad.reshape(1, _VPAD), bias_pad.reshape(_VPAD, 1))
    return out[0, 0]


# speedup vs baseline: 121.1087x; 1.0756x over previous
import functools
import math

import jax
import jax.numpy as jnp
from jax import lax
from jax.experimental import pallas as pl
from jax.experimental.pallas import tpu as pltpu
from jax.experimental.pallas import tpu_sc as plsc

_VOCAB = 1000
_D = 128
_B, _N = 64, 32
_K = 100
_BN = _B * _N
_KPAD = 128
_LOGK = math.log(_K)

_NC, _NS = 2, 16
_NW = _NC * _NS
_TPW = _BN // _NW
_NPW = 8


def _sc_gather_body(tgt_hbm, noise_hbm, emb_hbm, rows_out,
                    idx_v, rows_v, nidx_v, nrows_v, sem, nsem):
    wid = lax.axis_index("s") * _NC + lax.axis_index("c")
    base = wid * _TPW
    pltpu.sync_copy(tgt_hbm.at[pl.ds(base, _TPW)], idx_v)
    tcopy = pltpu.async_copy(emb_hbm.at[idx_v], rows_v, sem)
    is_noise_worker = wid < _KPAD // _NPW

    @pl.when(is_noise_worker)
    def _():
        pltpu.sync_copy(noise_hbm.at[pl.ds(wid * _NPW, _NPW)], nidx_v)
        pltpu.async_copy(emb_hbm.at[nidx_v], nrows_v, nsem)

    tcopy.wait()
    pltpu.sync_copy(rows_v, rows_out.at[pl.ds(base, _TPW)])

    @pl.when(is_noise_worker)
    def _():
        pltpu.make_async_copy(emb_hbm.at[nidx_v], nrows_v, nsem).wait()
        pltpu.sync_copy(nrows_v, rows_out.at[pl.ds(_BN + wid * _NPW, _NPW)])


@functools.cache
def _make_sc_gather():
  return pl.kernel(
    _sc_gather_body,
    out_type=jax.ShapeDtypeStruct((_BN + _KPAD, _D), jnp.float32),
    mesh=plsc.VectorSubcoreMesh(core_axis_name="c", subcore_axis_name="s",
                                num_cores=_NC, num_subcores=_NS),
    scratch_types=[
        pltpu.VMEM((_TPW,), jnp.int32),
        pltpu.VMEM((_TPW, _D), jnp.float32),
        pltpu.VMEM((_NPW,), jnp.int32),
        pltpu.VMEM((_NPW, _D), jnp.float32),
        pltpu.SemaphoreType.DMA,
        pltpu.SemaphoreType.DMA,
    ],
  )


def _tc_body(x_ref, rows_ref, tgt_ref, nz_ref, brow_ref, out_ref):
    x = x_ref[...]
    te = rows_ref[: _BN, :]
    ne = rows_ref[_BN:, :]
    brow = brow_ref[...]
    tgt = tgt_ref[...]
    vid_t = lax.broadcasted_iota(jnp.int32, (_BN, _VOCAB), 1)
    bt = jnp.sum(jnp.where(vid_t == tgt, brow, 0.0), axis=1, keepdims=True)
    nz = nz_ref[...]
    vid_n = lax.broadcasted_iota(jnp.int32, (_VOCAB, _KPAD), 0)
    onehot_n = jnp.where(vid_n == nz, 1.0, 0.0)
    bn = lax.dot_general(brow, onehot_n, (((1,), (0,)), ((), ())),
                         preferred_element_type=jnp.float32)

    t = jnp.sum(x * te, axis=1, keepdims=True) + bt - _LOGK
    s = lax.dot_general(x, ne, (((1,), (1,)), ((), ())),
                        preferred_element_type=jnp.float32)
    xn = s + bn - _LOGK
    sp_n = jnp.maximum(xn, 0.0) + jnp.log1p(jnp.exp(-jnp.abs(xn)))
    col = lax.broadcasted_iota(jnp.int32, sp_n.shape, 1)
    sp_n = jnp.where(col < _K, sp_n, 0.0)
    sp_t = jnp.maximum(-t, 0.0) + jnp.log1p(jnp.exp(-jnp.abs(t)))
    total = (jnp.sum(sp_n) + jnp.sum(sp_t)) * (1.0 / _BN)
    out_ref[...] = jnp.broadcast_to(total, (1, 1))


def kernel(target, input, emb, bias, noise_samples):
    tgt = target.reshape(_BN).astype(jnp.int32)
    noise = jnp.pad(noise_samples.reshape(_K).astype(jnp.int32),
                    (0, _KPAD - _K))
    x = input.reshape(_BN, _D)
    rows = _make_sc_gather()(tgt, noise, emb)
    out = pl.pallas_call(
        _tc_body,
        out_shape=jax.ShapeDtypeStruct((1, 1), jnp.float32),
    )(x, rows, tgt.reshape(_BN, 1), noise.reshape(1, _KPAD),
      bias.astype(jnp.float32).reshape(1, _VOCAB))
    return out[0, 0]


# speedup vs baseline: 127.7244x; 1.0546x over previous
import functools
import math

import jax
import jax.numpy as jnp
from jax import lax
from jax.experimental import pallas as pl
from jax.experimental.pallas import tpu as pltpu
from jax.experimental.pallas import tpu_sc as plsc

_VOCAB = 1000
_D = 128
_B, _N = 64, 32
_K = 100
_BN = _B * _N
_NPAD = 104
_LOGK = math.log(_K)

_NC, _NS = 2, 16
_NW = _NC * _NS
_TPW = _BN // _NW
_NPW = 8


def _sc_gather_body(tgt_hbm, noise_hbm, emb_hbm, rows_out,
                    idx_v, rows_v, nidx_v, nrows_v, sem, nsem):
    wid = lax.axis_index("s") * _NC + lax.axis_index("c")
    base = wid * _TPW
    pltpu.sync_copy(tgt_hbm.at[pl.ds(base, _TPW)], idx_v)
    pltpu.async_copy(emb_hbm.at[idx_v], rows_v, sem)

    nfull = _K // _NPW
    is_full = wid < nfull
    is_tail = wid == nfull

    @pl.when(is_full)
    def _():
        pltpu.sync_copy(noise_hbm.at[pl.ds(wid * _NPW, _NPW)], nidx_v)
        pltpu.async_copy(emb_hbm.at[nidx_v], nrows_v, nsem)

    @pl.when(is_tail)
    def _():
        pltpu.sync_copy(noise_hbm.at[pl.ds(nfull * _NPW, 4)],
                        nidx_v.at[pl.ds(0, 4)])
        pltpu.async_copy(emb_hbm.at[nidx_v.at[pl.ds(0, 4)]],
                         nrows_v.at[pl.ds(0, 4)], nsem)

    pltpu.make_async_copy(emb_hbm.at[idx_v], rows_v, sem).wait()
    pltpu.sync_copy(rows_v, rows_out.at[pl.ds(base, _TPW)])

    @pl.when(is_full)
    def _():
        pltpu.make_async_copy(emb_hbm.at[nidx_v], nrows_v, nsem).wait()
        pltpu.sync_copy(nrows_v, rows_out.at[pl.ds(_BN + wid * _NPW, _NPW)])

    @pl.when(is_tail)
    def _():
        pltpu.make_async_copy(emb_hbm.at[nidx_v.at[pl.ds(0, 4)]],
                              nrows_v.at[pl.ds(0, 4)], nsem).wait()
        pltpu.sync_copy(nrows_v.at[pl.ds(0, 4)],
                        rows_out.at[pl.ds(_BN + nfull * _NPW, 4)])


@functools.cache
def _make_sc_gather():
  return pl.kernel(
    _sc_gather_body,
    out_type=jax.ShapeDtypeStruct((_BN + _NPAD, _D), jnp.float32),
    mesh=plsc.VectorSubcoreMesh(core_axis_name="c", subcore_axis_name="s",
                                num_cores=_NC, num_subcores=_NS),
    scratch_types=[
        pltpu.VMEM((_TPW,), jnp.int32),
        pltpu.VMEM((_TPW, _D), jnp.float32),
        pltpu.VMEM((_NPW,), jnp.int32),
        pltpu.VMEM((_NPW, _D), jnp.float32),
        pltpu.SemaphoreType.DMA,
        pltpu.SemaphoreType.DMA,
    ],
  )


def _tc_body(x_ref, rows_ref, tgt_ref, nz_ref, bias_ref, out_ref):
    x3 = x_ref[...]
    x2 = x3.reshape(_BN, _D)
    te3 = rows_ref[: _BN, :].reshape(_B, _N, _D)
    ne = rows_ref[_BN:, :]
    bias = bias_ref[...]
    tgt = tgt_ref[...]
    vid_t = lax.broadcasted_iota(jnp.int32, (_B, _N, _VOCAB), 2)
    bt = jnp.sum(jnp.where(vid_t == tgt[:, :, None], bias, 0.0), axis=2)
    nz = nz_ref[...].reshape(1, _K)
    vid_n = lax.broadcasted_iota(jnp.int32, (_VOCAB, _K), 0)
    onehot_n = jnp.where(vid_n == nz, 1.0, 0.0)
    bn = lax.dot_general(bias.reshape(1, _VOCAB), onehot_n,
                         (((1,), (0,)), ((), ())),
                         preferred_element_type=jnp.float32)

    t = jnp.sum(x3 * te3, axis=2) + bt - _LOGK
    s = lax.dot_general(x2, ne, (((1,), (1,)), ((), ())),
                        preferred_element_type=jnp.float32)[:, : _K]
    xn = s + bn - _LOGK
    sp_n = jnp.maximum(xn, 0.0) + jnp.log1p(jnp.exp(-jnp.abs(xn)))
    sp_t = jnp.maximum(-t, 0.0) + jnp.log1p(jnp.exp(-jnp.abs(t)))
    total = (jnp.sum(sp_n) + jnp.sum(sp_t)) * (1.0 / _BN)
    out_ref[...] = jnp.broadcast_to(total, (1, 1))


def kernel(target, input, emb, bias, noise_samples):
    tgt = target.reshape(_BN).astype(jnp.int32)
    noise = noise_samples.reshape(_K).astype(jnp.int32)
    rows = _make_sc_gather()(tgt, noise, emb)
    out = pl.pallas_call(
        _tc_body,
        out_shape=jax.ShapeDtypeStruct((1, 1), jnp.float32),
    )(input, rows, target, noise_samples.reshape(_K),
      bias.astype(jnp.float32))
    return out[0, 0]
